# Initial kernel scaffold; baseline (speedup 1.0000x reference)
import jax
import jax.numpy as jnp
from jax.experimental import pallas as pl


def kernel(x, edge_index, edge_attr, batch, params):
    raise NotImplementedError("write your pallas kernel here")


# speedup vs baseline: 1.9778x; 1.9778x over previous
import functools

import jax
import jax.numpy as jnp
from jax import lax
from jax.experimental import pallas as pl
from jax.experimental.pallas import tpu as pltpu
from jax.experimental.pallas import tpu_sc as plsc

HC = 128
ED = 16
G = 512
QW = 256
BLK = 2000


def _proj_body(h_ref, wcat_ref, bcat_ref, we_ref, qcat_ref, kv_ref, skip_ref):
    h = h_ref[...]
    p = jnp.dot(h, wcat_ref[...], preferred_element_type=jnp.float32) + bcat_ref[...]
    q = p[:, :HC] * (1.0 / jnp.sqrt(jnp.float32(HC)))
    qe = jnp.dot(q, we_ref[...], preferred_element_type=jnp.float32)
    pad = jnp.zeros((q.shape[0], QW - HC - ED), jnp.float32)
    qcat_ref[...] = jnp.concatenate([q, qe, pad], axis=1)
    kv_ref[...] = p[:, HC:3 * HC]
    skip_ref[...] = p[:, 3 * HC:4 * HC]


def _proj(h, wcat, bcat, we):
    n = h.shape[0]
    grid = n // BLK
    return pl.pallas_call(
        _proj_body,
        grid=(grid,),
        in_specs=[
            pl.BlockSpec((BLK, HC), lambda i: (i, 0)),
            pl.BlockSpec((HC, 4 * HC), lambda i: (0, 0)),
            pl.BlockSpec((1, 4 * HC), lambda i: (0, 0)),
            pl.BlockSpec((HC, ED), lambda i: (0, 0)),
        ],
        out_specs=[
            pl.BlockSpec((BLK, QW), lambda i: (i, 0)),
            pl.BlockSpec((BLK, 2 * HC), lambda i: (i, 0)),
            pl.BlockSpec((BLK, HC), lambda i: (i, 0)),
        ],
        out_shape=[
            jax.ShapeDtypeStruct((n, QW), jnp.float32),
            jax.ShapeDtypeStruct((n, 2 * HC), jnp.float32),
            jax.ShapeDtypeStruct((n, HC), jnp.float32),
        ],
    )(h, wcat, bcat, we)


@functools.lru_cache(maxsize=None)
def _make_edge_call(n, e):
    info = plsc.get_sparse_core_info()
    nc, ns = 1, info.num_subcores
    nw = nc * ns
    ept = e // nw
    ch = 32
    nchunk = ept // ch
    rpt = (n // ns) & ~7
    tail = n - ns * rpt
    zr = 104
    nz = rpt // zr
    nwrow = (n // 4 + ns * 8 - 1) // (ns * 8) * (ns * 8)
    wpt = nwrow // ns
    wz = 80
    nwz = wpt // wz
    assert nz * zr == rpt and tail <= zr and tail % 8 == 0 and nwz * wz == wpt
    mesh = plsc.VectorSubcoreMesh(core_axis_name="c", subcore_axis_name="s",
                                  num_cores=nc)

    @functools.partial(
        pl.kernel,
        mesh=mesh,
        out_type=[
            jax.ShapeDtypeStruct((n, HC), jnp.float32),
            jax.ShapeDtypeStruct((nwrow, HC), jnp.float32),
        ],
        scratch_types=[
            pltpu.VMEM((ch,), jnp.int32),
            pltpu.VMEM((ch,), jnp.int32),
            pltpu.VMEM((ch,), jnp.int32),
            pltpu.VMEM((ch + 16,), jnp.int32),
            pltpu.VMEM((ch, ED), jnp.float32),
            pltpu.VMEM((ch, QW), jnp.float32),
            pltpu.VMEM((ch, 2 * HC), jnp.float32),
            pltpu.VMEM((ch, HC), jnp.float32),
            pltpu.VMEM((ch, HC), jnp.float32),
            pltpu.VMEM_SHARED((n, HC), jnp.float32),
            pltpu.VMEM_SHARED((nwrow, HC), jnp.float32),
            pltpu.SemaphoreType.DMA,
            pltpu.SemaphoreType.DMA,
        ],
    )
    def edge_kernel(qcat_hbm, kv_hbm, ea_hbm, src_hbm, dst_hbm, zeros_hbm,
                    outv_hbm, outw_hbm,
                    srci, dsti, wdxi, dstp, eav, qv, kvv, rowv, roww,
                    accv, accw, sem1, sem2):
        s = lax.axis_index("s")
        zeros16 = jnp.zeros((16,), jnp.float32)

        pltpu.sync_copy(zeros_hbm.at[pl.ds(s * rpt, rpt), :],
                        accv.at[pl.ds(s * rpt, rpt), :])
        pltpu.sync_copy(zeros_hbm.at[pl.ds(0, wpt), :],
                        accw.at[pl.ds(s * wpt, wpt), :])

        @pl.when(s == ns - 1)
        def _():
            pltpu.sync_copy(zeros_hbm.at[pl.ds(0, tail), :],
                            accv.at[pl.ds(ns * rpt, tail), :])

        plsc.subcore_barrier()

        lane = lax.broadcasted_iota(jnp.int32, (16,), 0)
        sone = jnp.where(lane == 0, 1.0, 0.0).astype(jnp.float32)
        bfly = [jnp.bitwise_xor(lane, d) for d in (1, 2, 4, 8)]
        gdn = lax.GatherDimensionNumbers(
            offset_dims=(), collapsed_slice_dims=(0,), start_index_map=(0,))

        def lane_shuf(x, idx):
            return lax.gather(x, idx[:, None], gdn, (1,),
                              mode=lax.GatherScatterMode.PROMISE_IN_BOUNDS)

        base = s * ept

        def chunk_body(i, carry):
            eb = base + i * ch
            pltpu.sync_copy(src_hbm.at[pl.ds(eb, ch)], srci)
            pltpu.sync_copy(dst_hbm.at[pl.ds(eb, ch)], dsti)
            pltpu.sync_copy(dst_hbm.at[pl.ds(eb, ch)], dstp.at[pl.ds(0, ch)])
            pltpu.sync_copy(ea_hbm.at[pl.ds(eb, ch), :], eav)
            cp1 = pltpu.async_copy(qcat_hbm.at[dsti], qv, sem1)
            cp2 = pltpu.async_copy(kv_hbm.at[srci], kvv, sem2)

            def wdx_body(t, carry2):
                dv = dsti[pl.ds(t * 16, 16)]
                wdxi[pl.ds(t * 16, 16)] = lax.shift_right_logical(dv, 2)
                return carry2

            lax.fori_loop(0, ch // 16, wdx_body, 0)
            cp1.wait()
            cp2.wait()

            def edge_body(j, carry2):
                acc = qv[j, pl.ds(0, 16)] * kvv[j, pl.ds(0, 16)]
                for t in range(1, 8):
                    acc = acc + qv[j, pl.ds(t * 16, 16)] * kvv[j, pl.ds(t * 16, 16)]
                acc = acc + qv[j, pl.ds(HC, 16)] * eav[j, :]
                for bidx in bfly:
                    acc = acc + lane_shuf(acc, bidx)
                ex = jnp.exp(acc)
                for t in range(8):
                    rowv[j, pl.ds(t * 16, 16)] = ex * kvv[j, pl.ds(HC + t * 16, 16)]
                for m in range(4):
                    roww[j, pl.ds(m * 32, 16)] = zeros16
                    roww[j, pl.ds(m * 32 + 16, 16)] = zeros16
                d4 = (dstp[pl.ds(j, 16)][0] & 3) * 32
                roww[j, pl.ds(d4, 16)] = ex * eav[j, :]
                roww[j, pl.ds(d4 + 16, 16)] = ex * sone
                return carry2

            lax.fori_loop(0, ch, edge_body, 0)
            pltpu.sync_copy(rowv, accv.at[dsti], add=True)
            pltpu.sync_copy(roww, accw.at[wdxi], add=True)
            return carry

        lax.fori_loop(0, nchunk, chunk_body, 0)
        plsc.subcore_barrier()
        pltpu.sync_copy(accv.at[pl.ds(s * rpt, rpt), :],
                        outv_hbm.at[pl.ds(s * rpt, rpt), :])
        pltpu.sync_copy(accw.at[pl.ds(s * wpt, wpt), :],
                        outw_hbm.at[pl.ds(s * wpt, wpt), :])

        @pl.when(s == ns - 1)
        def _():
            r0 = ns * rpt
            pltpu.sync_copy(accv.at[pl.ds(r0, tail), :],
                            outv_hbm.at[pl.ds(r0, tail), :])

    return edge_kernel


def _fin_body(av0_ref, aw0_ref, skip_ref, wet_ref, h_ref):
    attn = av0_ref[...]
    aw = aw0_ref[...]
    wv = aw[:, :ED]
    sden = aw[:, ED:ED + 1]
    h = (attn + jnp.dot(wv, wet_ref[...], preferred_element_type=jnp.float32)) \
        / (sden + 1e-16) + skip_ref[...]
    h_ref[...] = jnp.where(h > 0, h, 0.01 * h)


def _fin(av0, aw0, skip, wet):
    n = skip.shape[0]
    grid = n // BLK
    return pl.pallas_call(
        _fin_body,
        grid=(grid,),
        in_specs=[
            pl.BlockSpec((BLK, HC), lambda i: (i, 0)),
            pl.BlockSpec((BLK, 32), lambda i: (i, 0)),
            pl.BlockSpec((BLK, HC), lambda i: (i, 0)),
            pl.BlockSpec((ED, HC), lambda i: (0, 0)),
        ],
        out_specs=pl.BlockSpec((BLK, HC), lambda i: (i, 0)),
        out_shape=jax.ShapeDtypeStruct((n, HC), jnp.float32),
    )(av0, aw0, skip, wet)


def _pool_body(h_ref, b_ref, w1c_ref, hp_ref, hp1_ref):
    i = pl.program_id(0)

    @pl.when(i == 0)
    def _():
        hp_ref[...] = jnp.zeros_like(hp_ref)

    oh = (b_ref[...] == lax.broadcasted_iota(jnp.int32, (BLK, G), 1)
          ).astype(jnp.float32)
    hp_ref[...] += lax.dot_general(
        oh, h_ref[...], (((0,), (0,)), ((), ())),
        preferred_element_type=jnp.float32)

    @pl.when(i == pl.num_programs(0) - 1)
    def _():
        hp1_ref[...] = jnp.dot(hp_ref[...], w1c_ref[...],
                               preferred_element_type=jnp.float32)


def _pool(h, b2d, w1c):
    n = h.shape[0]
    grid = n // BLK
    sc = w1c.shape[1]
    return pl.pallas_call(
        _pool_body,
        grid=(grid,),
        in_specs=[
            pl.BlockSpec((BLK, HC), lambda i: (i, 0)),
            pl.BlockSpec((BLK, 1), lambda i: (i, 0)),
            pl.BlockSpec((HC, sc), lambda i: (0, 0)),
        ],
        out_specs=[
            pl.BlockSpec((G, HC), lambda i: (0, 0)),
            pl.BlockSpec((G, sc), lambda i: (0, 0)),
        ],
        out_shape=[
            jax.ShapeDtypeStruct((G, HC), jnp.float32),
            jax.ShapeDtypeStruct((G, sc), jnp.float32),
        ],
    )(h, b2d, w1c)


def _mlp_body(h1_ref, h2_ref, b_ref, hp1_ref, w1ab_ref, b1_ref,
              w2_ref, b2_ref, w3_ref, b3_ref, wf_ref, bf_ref, o_ref):
    hcat = jnp.concatenate([h1_ref[...], h2_ref[...]], axis=1)
    z = jnp.dot(hcat, w1ab_ref[...], preferred_element_type=jnp.float32)
    oh = (b_ref[...] == lax.broadcasted_iota(jnp.int32, (BLK, G), 1)
          ).astype(jnp.float32)
    z = z + jnp.dot(oh, hp1_ref[...], preferred_element_type=jnp.float32) \
        + b1_ref[...]
    t = jnp.dot(z, w2_ref[...], preferred_element_type=jnp.float32) + b2_ref[...]
    t = jnp.where(t > 0, t, 0.01 * t)
    t = jnp.dot(t, w3_ref[...], preferred_element_type=jnp.float32) + b3_ref[...]
    t = jnp.where(t > 0, t, 0.01 * t)
    o = jnp.dot(t, wf_ref[...], preferred_element_type=jnp.float32) + bf_ref[...]
    o_ref[...] = jax.nn.sigmoid(o)


def _mlp(h1, h2, b2d, hp1, w1ab, b1, w2t, b2r, w3t, b3r, wft, bfr):
    n = h1.shape[0]
    grid = n // BLK
    sc = w1ab.shape[1]
    return pl.pallas_call(
        _mlp_body,
        grid=(grid,),
        in_specs=[
            pl.BlockSpec((BLK, HC), lambda i: (i, 0)),
            pl.BlockSpec((BLK, HC), lambda i: (i, 0)),
            pl.BlockSpec((BLK, 1), lambda i: (i, 0)),
            pl.BlockSpec((G, sc), lambda i: (0, 0)),
            pl.BlockSpec((2 * HC, sc), lambda i: (0, 0)),
            pl.BlockSpec((1, sc), lambda i: (0, 0)),
            pl.BlockSpec((sc, sc), lambda i: (0, 0)),
            pl.BlockSpec((1, sc), lambda i: (0, 0)),
            pl.BlockSpec((sc, sc), lambda i: (0, 0)),
            pl.BlockSpec((1, sc), lambda i: (0, 0)),
            pl.BlockSpec((sc, 1), lambda i: (0, 0)),
            pl.BlockSpec((1, 1), lambda i: (0, 0)),
        ],
        out_specs=pl.BlockSpec((BLK, 1), lambda i: (i, 0)),
        out_shape=jax.ShapeDtypeStruct((n, 1), jnp.float32),
    )(h1, h2, b2d, hp1, w1ab, b1, w2t, b2r, w3t, b3r, wft, bfr)


def kernel(x, edge_index, edge_attr, batch, params):
    n = x.shape[0]
    e = edge_index.shape[1]
    src = edge_index[0]
    dst = edge_index[1]
    b2d = batch.reshape(n, 1)
    zrows = jnp.zeros((n, HC), jnp.float32)
    edge_call = _make_edge_call(n, e)

    h = x
    hs = []
    for li, name in enumerate(("conv1", "conv_l0", "conv_l1")):
        p = params[name]
        wcat = jnp.concatenate([p["Wq"], p["Wk"], p["Wv"], p["Ws"]], axis=0).T
        bcat = jnp.concatenate([p["bq"], p["bk"], p["bv"], p["bs"]]).reshape(1, -1)
        qcat, kv, skip = _proj(h, wcat, bcat, p["We"])
        accv, accw = edge_call(qcat, kv, edge_attr, src, dst, zrows)
        aw = accw.reshape(-1, 32)[:n, :]
        h = _fin(accv, aw, skip, p["We"].T)
        if li > 0:
            hs.append(h)

    h1, h2 = hs
    w1, b1 = params["cls1"]
    w2, b2 = params["cls_l0"]
    w3, b3 = params["cls_l1"]
    wf, bf = params["final"]
    _, hp1 = _pool(h2, b2d, w1[:, 2 * HC:].T)
    return _mlp(h1, h2, b2d, hp1,
                w1[:, :2 * HC].T, b1.reshape(1, -1),
                w2.T, b2.reshape(1, -1),
                w3.T, b3.reshape(1, -1),
                wf.T, bf.reshape(1, 1))


# speedup vs baseline: 5.3438x; 2.7019x over previous
import functools

import jax
import jax.numpy as jnp
import numpy as np
from jax import lax
from jax.experimental import pallas as pl
from jax.experimental.pallas import tpu as pltpu
from jax.experimental.pallas import tpu_sc as plsc

HC = 128
ED = 16
G = 512
QW = 256
BLK = 2000

def _bf16_bits(x):
    u = lax.bitcast_convert_type(x, jnp.int32)
    r = u + 0x7FFF + (lax.shift_right_logical(u, 16) & 1)
    return lax.shift_right_logical(r, 16)


def _pack_halves(x):
    m = x.shape[1] // 2
    r = _bf16_bits(x)
    return r[:, :m] | lax.shift_left(r[:, m:], 16)


def _proj_body(h_ref, wcat_ref, bcat_ref, we_ref, qcat_ref, kv_ref, skip_ref):
    h = h_ref[...]
    p = jnp.dot(h, wcat_ref[...], preferred_element_type=jnp.float32) + bcat_ref[...]
    q = p[:, :HC] * (1.0 / jnp.sqrt(jnp.float32(HC)))
    qei = jnp.dot(q, we_ref[...], preferred_element_type=jnp.float32)
    qpk = _pack_halves(q)
    qepk = _bf16_bits(qei)
    pad = jnp.zeros((q.shape[0], QW // 2 - HC // 2 - ED), jnp.int32)
    qcat_ref[...] = jnp.concatenate([qpk, qepk, pad], axis=1)
    kvf = p[:, HC:3 * HC]
    kv_ref[...] = jnp.concatenate(
        [_pack_halves(kvf[:, :HC]), _pack_halves(kvf[:, HC:])], axis=1)
    skip_ref[...] = p[:, 3 * HC:4 * HC]


def _proj(h, wcat, bcat, we):
    n = h.shape[0]
    grid = n // BLK
    return pl.pallas_call(
        _proj_body,
        grid=(grid,),
        in_specs=[
            pl.BlockSpec((BLK, HC), lambda i: (i, 0)),
            pl.BlockSpec((HC, 4 * HC), lambda i: (0, 0)),
            pl.BlockSpec((1, 4 * HC), lambda i: (0, 0)),
            pl.BlockSpec((HC, ED), lambda i: (0, 0)),
        ],
        out_specs=[
            pl.BlockSpec((BLK, QW // 2), lambda i: (i, 0)),
            pl.BlockSpec((BLK, HC), lambda i: (i, 0)),
            pl.BlockSpec((BLK, HC), lambda i: (i, 0)),
        ],
        out_shape=[
            jax.ShapeDtypeStruct((n, QW // 2), jnp.int32),
            jax.ShapeDtypeStruct((n, HC), jnp.int32),
            jax.ShapeDtypeStruct((n, HC), jnp.float32),
        ],
    )(h, wcat, bcat, we)


@functools.lru_cache(maxsize=None)
def _make_edge_call(n, e):
    info = plsc.get_sparse_core_info()
    ns = info.num_subcores
    ept = e // ns
    ch = 16
    nchunk = ept // ch
    kb = 50
    nwrow = (n // 4 + ns * 8 - 1) // (ns * 8) * (ns * 8)
    nrow = n + nwrow
    rpt = (nrow // ns) & ~7
    tail = nrow - ns * rpt
    assert nchunk % kb == 0 and tail % 8 == 0 and ch % 16 == 0
    mesh = plsc.VectorSubcoreMesh(core_axis_name="c", subcore_axis_name="s",
                                  num_cores=1)

    @functools.partial(
        pl.kernel,
        mesh=mesh,
        out_type=jax.ShapeDtypeStruct((nrow, HC), jnp.float32),
        scratch_types=[
            pltpu.VMEM((kb * ch,), jnp.int32),
            pltpu.VMEM((kb * ch,), jnp.int32),
            pltpu.VMEM((ch, QW // 2), jnp.int32),
            pltpu.VMEM((ch, QW // 2), jnp.int32),
            pltpu.VMEM((ch, HC), jnp.int32),
            pltpu.VMEM((ch, HC), jnp.int32),
            pltpu.VMEM((ch, ED), jnp.float32),
            pltpu.VMEM((ch, ED), jnp.float32),
            pltpu.VMEM((2 * ch, HC), jnp.float32),
            pltpu.VMEM((2 * ch,), jnp.int32),
            pltpu.VMEM((ch + 16,), jnp.int32),
            pltpu.VMEM_SHARED((nrow, HC), jnp.float32),
            pltpu.SemaphoreType.DMA,
            pltpu.SemaphoreType.DMA,
            pltpu.SemaphoreType.DMA,
        ],
    )
    def edge_kernel(qcat_hbm, kv_hbm, ea_hbm, src2_hbm, dst2_hbm, zeros_hbm,
                    out_hbm,
                    srcb, dstb, qv0, qv1, kv0, kv1, ea0, ea1,
                    rows2, idx2, d4p, accB, semg0, semg1, sems):
        s = lax.axis_index("s")
        zeros16 = jnp.zeros((16,), jnp.float32)
        lane = lax.broadcasted_iota(jnp.int32, (16,), 0)
        sone = jnp.where(lane == 0, 1.0, 0.0).astype(jnp.float32)
        bfly = [jnp.bitwise_xor(lane, d) for d in (1, 2, 4, 8)]
        gdn = lax.GatherDimensionNumbers(
            offset_dims=(), collapsed_slice_dims=(0,), start_index_map=(0,))

        def lane_shuf(x, idx):
            return lax.gather(x, idx[:, None], gdn, (1,),
                              mode=lax.GatherScatterMode.PROMISE_IN_BOUNDS)

        def unpk(u):
            lo = lax.bitcast_convert_type(lax.shift_left(u, 16), jnp.float32)
            hi = lax.bitcast_convert_type(
                jnp.bitwise_and(u, jnp.int32(-65536)), jnp.float32)
            return lo, hi

        ebase = s * ept

        pltpu.sync_copy(zeros_hbm.at[pl.ds(s * rpt, rpt), :],
                        accB.at[pl.ds(s * rpt, rpt), :])

        @pl.when(s == ns - 1)
        def _():
            pltpu.sync_copy(zeros_hbm.at[pl.ds(ns * rpt, tail), :],
                            accB.at[pl.ds(ns * rpt, tail), :])

        plsc.subcore_barrier()

        def load_block(bi):
            r0 = ebase + bi * (kb * ch)
            pltpu.sync_copy(src2_hbm.at[pl.ds(r0, kb * ch)], srcb)
            pltpu.sync_copy(dst2_hbm.at[pl.ds(r0, kb * ch)], dstb)

        def issue_gathers(c, qvb, kvb, eab, semg):
            jn = lax.rem(c, kb) * ch
            pltpu.async_copy(qcat_hbm.at[dstb.at[pl.ds(jn, ch)]], qvb, semg)
            pltpu.async_copy(kv_hbm.at[srcb.at[pl.ds(jn, ch)]], kvb, semg)
            pltpu.async_copy(ea_hbm.at[pl.ds(ebase + c * ch, ch), :], eab, semg)

        def drain_gathers(c, qvb, kvb, eab, semg):
            jc = lax.rem(c, kb) * ch
            pltpu.make_async_copy(
                qcat_hbm.at[dstb.at[pl.ds(jc, ch)]], qvb, semg).wait()
            pltpu.make_async_copy(
                kv_hbm.at[srcb.at[pl.ds(jc, ch)]], kvb, semg).wait()
            pltpu.make_async_copy(
                ea_hbm.at[pl.ds(ebase + c * ch, ch), :], eab, semg).wait()

        def mk_edge_body(qvb, kvb, eab):
            def edge_body(j, carry):
                acc = None
                for t in range(4):
                    uq0, uq1 = unpk(qvb[j, pl.ds(16 * t, 16)])
                    uk0, uk1 = unpk(kvb[j, pl.ds(16 * t, 16)])
                    term = uq0 * uk0 + uq1 * uk1
                    acc = term if acc is None else acc + term
                ue0, _ = unpk(qvb[j, pl.ds(64, 16)])
                acc = acc + ue0 * eab[j, :]
                for bidx in bfly:
                    acc = acc + lane_shuf(acc, bidx)
                ex = jnp.exp(acc)
                for t in range(4):
                    uv0, uv1 = unpk(kvb[j, pl.ds(64 + 16 * t, 16)])
                    rows2[j, pl.ds(16 * t, 16)] = ex * uv0
                    rows2[j, pl.ds(64 + 16 * t, 16)] = ex * uv1
                for m in range(4):
                    rows2[ch + j, pl.ds(m * 32, 16)] = zeros16
                    rows2[ch + j, pl.ds(m * 32 + 16, 16)] = zeros16
                d4 = d4p[pl.ds(j, 16)][0]
                rows2[ch + j, pl.ds(d4, 16)] = ex * eab[j, :]
                rows2[ch + j, pl.ds(d4 + 16, 16)] = ex * sone
                return carry
            return edge_body

        def do_chunk(c, qvb, kvb, eab, semg, qvn, kvn, ean, semgn):
            @pl.when(c > 0)
            def _():
                pltpu.make_async_copy(rows2, accB.at[idx2], sems).wait()

            jc = lax.rem(c, kb) * ch
            for g in range(ch // 16):
                dv = dstb[pl.ds(jc + 16 * g, 16)]
                idx2[pl.ds(16 * g, 16)] = dv
                idx2[pl.ds(ch + 16 * g, 16)] = \
                    n + lax.shift_right_logical(dv, 2)
                d4p[pl.ds(16 * g, 16)] = (dv & 3) * 32

            boundary = lax.rem(c + 1, kb) == 0
            not_last = c + 1 < nchunk

            @pl.when(jnp.logical_and(not_last, jnp.logical_not(boundary)))
            def _():
                issue_gathers(c + 1, qvn, kvn, ean, semgn)

            drain_gathers(c, qvb, kvb, eab, semg)

            @pl.when(jnp.logical_and(not_last, boundary))
            def _():
                load_block((c + 1) // kb)
                issue_gathers(c + 1, qvn, kvn, ean, semgn)

            lax.fori_loop(0, ch, mk_edge_body(qvb, kvb, eab), 0)
            pltpu.async_copy(rows2, accB.at[idx2], sems, add=True)

        load_block(0)
        issue_gathers(0, qv0, kv0, ea0, semg0)

        def loop_body(i, carry):
            c0 = i * 2
            do_chunk(c0, qv0, kv0, ea0, semg0, qv1, kv1, ea1, semg1)
            do_chunk(c0 + 1, qv1, kv1, ea1, semg1, qv0, kv0, ea0, semg0)
            return carry

        lax.fori_loop(0, nchunk // 2, loop_body, 0)
        if nchunk % 2:
            do_chunk(jnp.int32(nchunk - 1), qv0, kv0, ea0, semg0,
                     qv1, kv1, ea1, semg1)
        pltpu.make_async_copy(rows2, accB.at[idx2], sems).wait()

        plsc.subcore_barrier()
        pltpu.sync_copy(accB.at[pl.ds(s * rpt, rpt), :],
                        out_hbm.at[pl.ds(s * rpt, rpt), :])

        @pl.when(s == ns - 1)
        def _():
            pltpu.sync_copy(accB.at[pl.ds(ns * rpt, tail), :],
                            out_hbm.at[pl.ds(ns * rpt, tail), :])

    return edge_kernel


def _fin_body(av0_ref, aw0_ref, skip_ref, wet_ref, h_ref):
    attn = av0_ref[...]
    aw = aw0_ref[...]
    wv = aw[:, :ED]
    sden = aw[:, ED:ED + 1]
    h = (attn + jnp.dot(wv, wet_ref[...], preferred_element_type=jnp.float32)) \
        / (sden + 1e-16) + skip_ref[...]
    h_ref[...] = jnp.where(h > 0, h, 0.01 * h)


def _fin(av0, aw0, skip, wet):
    n = skip.shape[0]
    grid = n // BLK
    return pl.pallas_call(
        _fin_body,
        grid=(grid,),
        in_specs=[
            pl.BlockSpec((BLK, HC), lambda i: (i, 0)),
            pl.BlockSpec((BLK, 32), lambda i: (i, 0)),
            pl.BlockSpec((BLK, HC), lambda i: (i, 0)),
            pl.BlockSpec((ED, HC), lambda i: (0, 0)),
        ],
        out_specs=pl.BlockSpec((BLK, HC), lambda i: (i, 0)),
        out_shape=jax.ShapeDtypeStruct((n, HC), jnp.float32),
    )(av0, aw0, skip, wet)


def _pool_body(h_ref, b_ref, w1c_ref, hp_ref, hp1_ref):
    i = pl.program_id(0)

    @pl.when(i == 0)
    def _():
        hp_ref[...] = jnp.zeros_like(hp_ref)

    oh = (b_ref[...] == lax.broadcasted_iota(jnp.int32, (BLK, G), 1)
          ).astype(jnp.float32)
    hp_ref[...] += lax.dot_general(
        oh, h_ref[...], (((0,), (0,)), ((), ())),
        preferred_element_type=jnp.float32)

    @pl.when(i == pl.num_programs(0) - 1)
    def _():
        hp1_ref[...] = jnp.dot(hp_ref[...], w1c_ref[...],
                               preferred_element_type=jnp.float32)


def _pool(h, b2d, w1c):
    n = h.shape[0]
    grid = n // BLK
    sc = w1c.shape[1]
    return pl.pallas_call(
        _pool_body,
        grid=(grid,),
        in_specs=[
            pl.BlockSpec((BLK, HC), lambda i: (i, 0)),
            pl.BlockSpec((BLK, 1), lambda i: (i, 0)),
            pl.BlockSpec((HC, sc), lambda i: (0, 0)),
        ],
        out_specs=[
            pl.BlockSpec((G, HC), lambda i: (0, 0)),
            pl.BlockSpec((G, sc), lambda i: (0, 0)),
        ],
        out_shape=[
            jax.ShapeDtypeStruct((G, HC), jnp.float32),
            jax.ShapeDtypeStruct((G, sc), jnp.float32),
        ],
    )(h, b2d, w1c)


def _mlp_body(h1_ref, h2_ref, b_ref, hp1_ref, w1ab_ref, b1_ref,
              w2_ref, b2_ref, w3_ref, b3_ref, wf_ref, bf_ref, o_ref):
    hcat = jnp.concatenate([h1_ref[...], h2_ref[...]], axis=1)
    z = jnp.dot(hcat, w1ab_ref[...], preferred_element_type=jnp.float32)
    oh = (b_ref[...] == lax.broadcasted_iota(jnp.int32, (BLK, G), 1)
          ).astype(jnp.float32)
    z = z + jnp.dot(oh, hp1_ref[...], preferred_element_type=jnp.float32) \
        + b1_ref[...]
    t = jnp.dot(z, w2_ref[...], preferred_element_type=jnp.float32) + b2_ref[...]
    t = jnp.where(t > 0, t, 0.01 * t)
    t = jnp.dot(t, w3_ref[...], preferred_element_type=jnp.float32) + b3_ref[...]
    t = jnp.where(t > 0, t, 0.01 * t)
    o = jnp.dot(t, wf_ref[...], preferred_element_type=jnp.float32) + bf_ref[...]
    o_ref[...] = jax.nn.sigmoid(o)


def _mlp(h1, h2, b2d, hp1, w1ab, b1, w2t, b2r, w3t, b3r, wft, bfr):
    n = h1.shape[0]
    grid = n // BLK
    sc = w1ab.shape[1]
    return pl.pallas_call(
        _mlp_body,
        grid=(grid,),
        in_specs=[
            pl.BlockSpec((BLK, HC), lambda i: (i, 0)),
            pl.BlockSpec((BLK, HC), lambda i: (i, 0)),
            pl.BlockSpec((BLK, 1), lambda i: (i, 0)),
            pl.BlockSpec((G, sc), lambda i: (0, 0)),
            pl.BlockSpec((2 * HC, sc), lambda i: (0, 0)),
            pl.BlockSpec((1, sc), lambda i: (0, 0)),
            pl.BlockSpec((sc, sc), lambda i: (0, 0)),
            pl.BlockSpec((1, sc), lambda i: (0, 0)),
            pl.BlockSpec((sc, sc), lambda i: (0, 0)),
            pl.BlockSpec((1, sc), lambda i: (0, 0)),
            pl.BlockSpec((sc, 1), lambda i: (0, 0)),
            pl.BlockSpec((1, 1), lambda i: (0, 0)),
        ],
        out_specs=pl.BlockSpec((BLK, 1), lambda i: (i, 0)),
        out_shape=jax.ShapeDtypeStruct((n, 1), jnp.float32),
    )(h1, h2, b2d, hp1, w1ab, b1, w2t, b2r, w3t, b3r, wft, bfr)


def kernel(x, edge_index, edge_attr, batch, params):
    n = x.shape[0]
    e = edge_index.shape[1]
    src2 = edge_index[0]
    dst2 = edge_index[1]
    b2d = batch.reshape(n, 1)
    edge_call = _make_edge_call(n, e)
    nwrow = (n // 4 + 16 * 8 - 1) // (16 * 8) * (16 * 8)
    zrows = jnp.zeros((n + nwrow, HC), jnp.float32)

    h = x
    hs = []
    for li, name in enumerate(("conv1", "conv_l0", "conv_l1")):
        p = params[name]
        wcat = jnp.concatenate([p["Wq"], p["Wk"], p["Wv"], p["Ws"]], axis=0).T
        bcat = jnp.concatenate([p["bq"], p["bk"], p["bv"], p["bs"]])
        qcat, kv, skip = _proj(h, wcat, bcat.reshape(1, -1), p["We"])
        accB = edge_call(qcat, kv, edge_attr, src2, dst2, zrows)
        aw = accB[n:].reshape(-1, 32)[:n, :]
        h = _fin(accB[:n], aw, skip, p["We"].T)
        if li > 0:
            hs.append(h)

    h1, h2 = hs
    w1, b1 = params["cls1"]
    w2, b2 = params["cls_l0"]
    w3, b3 = params["cls_l1"]
    wf, bf = params["final"]
    w1ab = w1[:, :2 * HC].T
    w1c = w1[:, 2 * HC:].T
    _, hp1 = _pool(h2, b2d, w1c)
    return _mlp(h1, h2, b2d, hp1,
                w1ab, b1.reshape(1, -1),
                w2.T, b2.reshape(1, -1),
                w3.T, b3.reshape(1, -1),
                wf.T, bf.reshape(1, 1))


# speedup vs baseline: 11.1753x; 2.0913x over previous
import functools

import jax
import jax.numpy as jnp
import numpy as np
from jax import lax
from jax.experimental import pallas as pl
from jax.experimental.pallas import tpu as pltpu
from jax.experimental.pallas import tpu_sc as plsc

HC = 128
ED = 16
G = 512
QW = 256
BLK = 2000

def _bf16_bits(x):
    u = lax.bitcast_convert_type(x, jnp.int32)
    r = u + 0x7FFF + (lax.shift_right_logical(u, 16) & 1)
    return lax.shift_right_logical(r, 16)


def _pack_halves(x):
    m = x.shape[1] // 2
    r = _bf16_bits(x)
    return r[:, :m] | lax.shift_left(r[:, m:], 16)


def _proj_body(h_ref, wcat_ref, bcat_ref, we_ref, qcat_ref, kv_ref, skip_ref):
    h = h_ref[...]
    p = jnp.dot(h, wcat_ref[...], preferred_element_type=jnp.float32) + bcat_ref[...]
    q = p[:, :HC] * (1.0 / jnp.sqrt(jnp.float32(HC)))
    qei = jnp.dot(q, we_ref[...], preferred_element_type=jnp.float32)
    qpk = _pack_halves(q)
    qepk = _bf16_bits(qei)
    pad = jnp.zeros((q.shape[0], QW // 2 - HC // 2 - ED), jnp.int32)
    qcat_ref[...] = jnp.concatenate([qpk, qepk, pad], axis=1)
    kvf = p[:, HC:3 * HC]
    kv_ref[...] = jnp.concatenate(
        [_pack_halves(kvf[:, :HC]), _pack_halves(kvf[:, HC:])], axis=1)
    skip_ref[...] = p[:, 3 * HC:4 * HC]


def _proj(h, wcat, bcat, we):
    n = h.shape[0]
    grid = n // BLK
    return pl.pallas_call(
        _proj_body,
        grid=(grid,),
        in_specs=[
            pl.BlockSpec((BLK, HC), lambda i: (i, 0)),
            pl.BlockSpec((HC, 4 * HC), lambda i: (0, 0)),
            pl.BlockSpec((1, 4 * HC), lambda i: (0, 0)),
            pl.BlockSpec((HC, ED), lambda i: (0, 0)),
        ],
        out_specs=[
            pl.BlockSpec((BLK, QW // 2), lambda i: (i, 0)),
            pl.BlockSpec((BLK, HC), lambda i: (i, 0)),
            pl.BlockSpec((BLK, HC), lambda i: (i, 0)),
        ],
        out_shape=[
            jax.ShapeDtypeStruct((n, QW // 2), jnp.int32),
            jax.ShapeDtypeStruct((n, HC), jnp.int32),
            jax.ShapeDtypeStruct((n, HC), jnp.float32),
        ],
    )(h, wcat, bcat, we)


@functools.lru_cache(maxsize=None)
def _make_edge_call(n, e):
    info = plsc.get_sparse_core_info()
    nc, ns = info.num_cores, info.num_subcores
    ept = e // (nc * ns)
    ch = 16
    nchunk = ept // ch
    kb = 25
    nwrow = (n // 4 + ns * 8 - 1) // (ns * 8) * (ns * 8)
    nrow = n + nwrow
    rpt = (nrow // ns) & ~7
    tail = nrow - ns * rpt
    assert nchunk % kb == 0 and tail % 8 == 0 and ch % 16 == 0
    mesh = plsc.VectorSubcoreMesh(core_axis_name="c", subcore_axis_name="s",
                                  num_cores=nc)

    @functools.partial(
        pl.kernel,
        mesh=mesh,
        out_type=jax.ShapeDtypeStruct((nc, nrow, HC), jnp.float32),
        scratch_types=[
            pltpu.VMEM((kb * ch,), jnp.int32),
            pltpu.VMEM((kb * ch,), jnp.int32),
            pltpu.VMEM((ch, QW // 2), jnp.int32),
            pltpu.VMEM((ch, QW // 2), jnp.int32),
            pltpu.VMEM((ch, HC), jnp.int32),
            pltpu.VMEM((ch, HC), jnp.int32),
            pltpu.VMEM((ch, ED), jnp.float32),
            pltpu.VMEM((ch, ED), jnp.float32),
            pltpu.VMEM((2 * ch, HC), jnp.float32),
            pltpu.VMEM((2 * ch,), jnp.int32),
            pltpu.VMEM((ch + 16,), jnp.int32),
            pltpu.VMEM_SHARED((nrow, HC), jnp.float32),
            pltpu.SemaphoreType.DMA,
            pltpu.SemaphoreType.DMA,
            pltpu.SemaphoreType.DMA,
        ],
    )
    def edge_kernel(qcat_hbm, kv_hbm, ea_hbm, src2_hbm, dst2_hbm, zeros_hbm,
                    out_hbm,
                    srcb, dstb, qv0, qv1, kv0, kv1, ea0, ea1,
                    rows2, idx2, d4p, accB, semg0, semg1, sems):
        cid = lax.axis_index("c")
        s = lax.axis_index("s")
        zeros16 = jnp.zeros((16,), jnp.float32)
        lane = lax.broadcasted_iota(jnp.int32, (16,), 0)
        sone = jnp.where(lane == 0, 1.0, 0.0).astype(jnp.float32)
        bfly = [jnp.bitwise_xor(lane, d) for d in (1, 2, 4, 8)]
        gdn = lax.GatherDimensionNumbers(
            offset_dims=(), collapsed_slice_dims=(0,), start_index_map=(0,))

        def lane_shuf(x, idx):
            return lax.gather(x, idx[:, None], gdn, (1,),
                              mode=lax.GatherScatterMode.PROMISE_IN_BOUNDS)

        def unpk(u):
            lo = lax.bitcast_convert_type(lax.shift_left(u, 16), jnp.float32)
            hi = lax.bitcast_convert_type(
                jnp.bitwise_and(u, jnp.int32(-65536)), jnp.float32)
            return lo, hi

        ebase = (s * nc + cid) * ept

        pltpu.sync_copy(zeros_hbm.at[pl.ds(s * rpt, rpt), :],
                        accB.at[pl.ds(s * rpt, rpt), :])

        @pl.when(s == ns - 1)
        def _():
            pltpu.sync_copy(zeros_hbm.at[pl.ds(ns * rpt, tail), :],
                            accB.at[pl.ds(ns * rpt, tail), :])

        plsc.subcore_barrier()

        def load_block(bi):
            r0 = ebase + bi * (kb * ch)
            pltpu.sync_copy(src2_hbm.at[pl.ds(r0, kb * ch)], srcb)
            pltpu.sync_copy(dst2_hbm.at[pl.ds(r0, kb * ch)], dstb)

        def issue_gathers(c, qvb, kvb, eab, semg):
            jn = lax.rem(c, kb) * ch
            pltpu.async_copy(qcat_hbm.at[dstb.at[pl.ds(jn, ch)]], qvb, semg)
            pltpu.async_copy(kv_hbm.at[srcb.at[pl.ds(jn, ch)]], kvb, semg)
            pltpu.async_copy(ea_hbm.at[pl.ds(ebase + c * ch, ch), :], eab, semg)

        def drain_gathers(c, qvb, kvb, eab, semg):
            jc = lax.rem(c, kb) * ch
            pltpu.make_async_copy(
                qcat_hbm.at[dstb.at[pl.ds(jc, ch)]], qvb, semg).wait()
            pltpu.make_async_copy(
                kv_hbm.at[srcb.at[pl.ds(jc, ch)]], kvb, semg).wait()
            pltpu.make_async_copy(
                ea_hbm.at[pl.ds(ebase + c * ch, ch), :], eab, semg).wait()

        def mk_edge_body(qvb, kvb, eab):
            def edge_body(j, carry):
                acc = None
                for t in range(4):
                    uq0, uq1 = unpk(qvb[j, pl.ds(16 * t, 16)])
                    uk0, uk1 = unpk(kvb[j, pl.ds(16 * t, 16)])
                    term = uq0 * uk0 + uq1 * uk1
                    acc = term if acc is None else acc + term
                ue0, _ = unpk(qvb[j, pl.ds(64, 16)])
                acc = acc + ue0 * eab[j, :]
                for bidx in bfly:
                    acc = acc + lane_shuf(acc, bidx)
                ex = jnp.exp(acc)
                for t in range(4):
                    uv0, uv1 = unpk(kvb[j, pl.ds(64 + 16 * t, 16)])
                    rows2[j, pl.ds(16 * t, 16)] = ex * uv0
                    rows2[j, pl.ds(64 + 16 * t, 16)] = ex * uv1
                for m in range(4):
                    rows2[ch + j, pl.ds(m * 32, 16)] = zeros16
                    rows2[ch + j, pl.ds(m * 32 + 16, 16)] = zeros16
                d4 = d4p[pl.ds(j, 16)][0]
                rows2[ch + j, pl.ds(d4, 16)] = ex * eab[j, :]
                rows2[ch + j, pl.ds(d4 + 16, 16)] = ex * sone
                return carry
            return edge_body

        def do_chunk(c, qvb, kvb, eab, semg, qvn, kvn, ean, semgn):
            @pl.when(c > 0)
            def _():
                pltpu.make_async_copy(rows2, accB.at[idx2], sems).wait()

            jc = lax.rem(c, kb) * ch
            for g in range(ch // 16):
                dv = dstb[pl.ds(jc + 16 * g, 16)]
                idx2[pl.ds(16 * g, 16)] = dv
                idx2[pl.ds(ch + 16 * g, 16)] = \
                    n + lax.shift_right_logical(dv, 2)
                d4p[pl.ds(16 * g, 16)] = (dv & 3) * 32

            boundary = lax.rem(c + 1, kb) == 0
            not_last = c + 1 < nchunk

            @pl.when(jnp.logical_and(not_last, jnp.logical_not(boundary)))
            def _():
                issue_gathers(c + 1, qvn, kvn, ean, semgn)

            drain_gathers(c, qvb, kvb, eab, semg)

            @pl.when(jnp.logical_and(not_last, boundary))
            def _():
                load_block((c + 1) // kb)
                issue_gathers(c + 1, qvn, kvn, ean, semgn)

            lax.fori_loop(0, ch, mk_edge_body(qvb, kvb, eab), 0)
            pltpu.async_copy(rows2, accB.at[idx2], sems, add=True)

        load_block(0)
        issue_gathers(0, qv0, kv0, ea0, semg0)

        def loop_body(i, carry):
            c0 = i * 2
            do_chunk(c0, qv0, kv0, ea0, semg0, qv1, kv1, ea1, semg1)
            do_chunk(c0 + 1, qv1, kv1, ea1, semg1, qv0, kv0, ea0, semg0)
            return carry

        lax.fori_loop(0, nchunk // 2, loop_body, 0)
        if nchunk % 2:
            do_chunk(jnp.int32(nchunk - 1), qv0, kv0, ea0, semg0,
                     qv1, kv1, ea1, semg1)
        pltpu.make_async_copy(rows2, accB.at[idx2], sems).wait()

        plsc.subcore_barrier()
        pltpu.sync_copy(accB.at[pl.ds(s * rpt, rpt), :],
                        out_hbm.at[cid, pl.ds(s * rpt, rpt), :])

        @pl.when(s == ns - 1)
        def _():
            pltpu.sync_copy(accB.at[pl.ds(ns * rpt, tail), :],
                            out_hbm.at[cid, pl.ds(ns * rpt, tail), :])

    return edge_kernel


def _fin_body(av0_ref, av1_ref, aw0_ref, aw1_ref, skip_ref, wet_ref, h_ref):
    attn = av0_ref[...] + av1_ref[...]
    aw = aw0_ref[...] + aw1_ref[...]
    wv = aw[:, :ED]
    sden = aw[:, ED:ED + 1]
    h = (attn + jnp.dot(wv, wet_ref[...], preferred_element_type=jnp.float32)) \
        / (sden + 1e-16) + skip_ref[...]
    h_ref[...] = jnp.where(h > 0, h, 0.01 * h)


def _fin(av0, av1, aw0, aw1, skip, wet):
    n = skip.shape[0]
    grid = n // BLK
    return pl.pallas_call(
        _fin_body,
        grid=(grid,),
        in_specs=[
            pl.BlockSpec((BLK, HC), lambda i: (i, 0)),
            pl.BlockSpec((BLK, HC), lambda i: (i, 0)),
            pl.BlockSpec((BLK, 32), lambda i: (i, 0)),
            pl.BlockSpec((BLK, 32), lambda i: (i, 0)),
            pl.BlockSpec((BLK, HC), lambda i: (i, 0)),
            pl.BlockSpec((ED, HC), lambda i: (0, 0)),
        ],
        out_specs=pl.BlockSpec((BLK, HC), lambda i: (i, 0)),
        out_shape=jax.ShapeDtypeStruct((n, HC), jnp.float32),
    )(av0, av1, aw0, aw1, skip, wet)


def _pool_body(h_ref, b_ref, w1c_ref, hp_ref, hp1_ref):
    i = pl.program_id(0)

    @pl.when(i == 0)
    def _():
        hp_ref[...] = jnp.zeros_like(hp_ref)

    oh = (b_ref[...] == lax.broadcasted_iota(jnp.int32, (BLK, G), 1)
          ).astype(jnp.float32)
    hp_ref[...] += lax.dot_general(
        oh, h_ref[...], (((0,), (0,)), ((), ())),
        preferred_element_type=jnp.float32)

    @pl.when(i == pl.num_programs(0) - 1)
    def _():
        hp1_ref[...] = jnp.dot(hp_ref[...], w1c_ref[...],
                               preferred_element_type=jnp.float32)


def _pool(h, b2d, w1c):
    n = h.shape[0]
    grid = n // BLK
    sc = w1c.shape[1]
    return pl.pallas_call(
        _pool_body,
        grid=(grid,),
        in_specs=[
            pl.BlockSpec((BLK, HC), lambda i: (i, 0)),
            pl.BlockSpec((BLK, 1), lambda i: (i, 0)),
            pl.BlockSpec((HC, sc), lambda i: (0, 0)),
        ],
        out_specs=[
            pl.BlockSpec((G, HC), lambda i: (0, 0)),
            pl.BlockSpec((G, sc), lambda i: (0, 0)),
        ],
        out_shape=[
            jax.ShapeDtypeStruct((G, HC), jnp.float32),
            jax.ShapeDtypeStruct((G, sc), jnp.float32),
        ],
    )(h, b2d, w1c)


def _mlp_body(h1_ref, h2_ref, b_ref, hp1_ref, w1ab_ref, b1_ref,
              w2_ref, b2_ref, w3_ref, b3_ref, wf_ref, bf_ref, o_ref):
    hcat = jnp.concatenate([h1_ref[...], h2_ref[...]], axis=1)
    z = jnp.dot(hcat, w1ab_ref[...], preferred_element_type=jnp.float32)
    oh = (b_ref[...] == lax.broadcasted_iota(jnp.int32, (BLK, G), 1)
          ).astype(jnp.float32)
    z = z + jnp.dot(oh, hp1_ref[...], preferred_element_type=jnp.float32) \
        + b1_ref[...]
    t = jnp.dot(z, w2_ref[...], preferred_element_type=jnp.float32) + b2_ref[...]
    t = jnp.where(t > 0, t, 0.01 * t)
    t = jnp.dot(t, w3_ref[...], preferred_element_type=jnp.float32) + b3_ref[...]
    t = jnp.where(t > 0, t, 0.01 * t)
    o = jnp.dot(t, wf_ref[...], preferred_element_type=jnp.float32) + bf_ref[...]
    o_ref[...] = jax.nn.sigmoid(o)


def _mlp(h1, h2, b2d, hp1, w1ab, b1, w2t, b2r, w3t, b3r, wft, bfr):
    n = h1.shape[0]
    grid = n // BLK
    sc = w1ab.shape[1]
    return pl.pallas_call(
        _mlp_body,
        grid=(grid,),
        in_specs=[
            pl.BlockSpec((BLK, HC), lambda i: (i, 0)),
            pl.BlockSpec((BLK, HC), lambda i: (i, 0)),
            pl.BlockSpec((BLK, 1), lambda i: (i, 0)),
            pl.BlockSpec((G, sc), lambda i: (0, 0)),
            pl.BlockSpec((2 * HC, sc), lambda i: (0, 0)),
            pl.BlockSpec((1, sc), lambda i: (0, 0)),
            pl.BlockSpec((sc, sc), lambda i: (0, 0)),
            pl.BlockSpec((1, sc), lambda i: (0, 0)),
            pl.BlockSpec((sc, sc), lambda i: (0, 0)),
            pl.BlockSpec((1, sc), lambda i: (0, 0)),
            pl.BlockSpec((sc, 1), lambda i: (0, 0)),
            pl.BlockSpec((1, 1), lambda i: (0, 0)),
        ],
        out_specs=pl.BlockSpec((BLK, 1), lambda i: (i, 0)),
        out_shape=jax.ShapeDtypeStruct((n, 1), jnp.float32),
    )(h1, h2, b2d, hp1, w1ab, b1, w2t, b2r, w3t, b3r, wft, bfr)


def kernel(x, edge_index, edge_attr, batch, params):
    n = x.shape[0]
    e = edge_index.shape[1]
    src2 = edge_index[0]
    dst2 = edge_index[1]
    b2d = batch.reshape(n, 1)
    edge_call = _make_edge_call(n, e)
    nwrow = (n // 4 + 16 * 8 - 1) // (16 * 8) * (16 * 8)
    zrows = jnp.zeros((n + nwrow, HC), jnp.float32)

    h = x
    hs = []
    for li, name in enumerate(("conv1", "conv_l0", "conv_l1")):
        p = params[name]
        wcat = jnp.concatenate([p["Wq"], p["Wk"], p["Wv"], p["Ws"]], axis=0).T
        bcat = jnp.concatenate([p["bq"], p["bk"], p["bv"], p["bs"]])
        qcat, kv, skip = _proj(h, wcat, bcat.reshape(1, -1), p["We"])
        accB = edge_call(qcat, kv, edge_attr, src2, dst2, zrows)
        aw0 = accB[0, n:].reshape(-1, 32)[:n, :]
        aw1 = accB[1, n:].reshape(-1, 32)[:n, :]
        h = _fin(accB[0, :n], accB[1, :n], aw0, aw1, skip, p["We"].T)
        if li > 0:
            hs.append(h)

    h1, h2 = hs
    w1, b1 = params["cls1"]
    w2, b2 = params["cls_l0"]
    w3, b3 = params["cls_l1"]
    wf, bf = params["final"]
    w1ab = w1[:, :2 * HC].T
    w1c = w1[:, 2 * HC:].T
    _, hp1 = _pool(h2, b2d, w1c)
    return _mlp(h1, h2, b2d, hp1,
                w1ab, b1.reshape(1, -1),
                w2.T, b2.reshape(1, -1),
                w3.T, b3.reshape(1, -1),
                wf.T, bf.reshape(1, 1))
